# baseline (speedup 1.0000x reference)
import functools
import jax
import jax.numpy as jnp
from jax import lax
from jax.experimental import pallas as pl
from jax.experimental.pallas import tpu as pltpu, tpu_sc as plsc


def kernel(x, pe):
    B, S, D = x.shape
    NC, NS = 2, 16
    NW = NC * NS
    N = B * S * D
    PER_W = N // NW
    CH = 16384
    N_CHUNKS = PER_W // CH
    S_ELEMS = S * D

    x_flat = x.reshape(N)
    pe_flat = pe[:S].reshape(S_ELEMS)

    mesh = plsc.VectorSubcoreMesh(core_axis_name="c", subcore_axis_name="s")

    @functools.partial(
        pl.kernel,
        mesh=mesh,
        out_type=jax.ShapeDtypeStruct((N,), jnp.float32),
        scratch_types=[
            pltpu.VMEM((2, CH), jnp.float32),
            pltpu.VMEM((2, CH), jnp.float32),
            pltpu.VMEM((2, CH), jnp.float32),
            pltpu.SemaphoreType.DMA,
            pltpu.SemaphoreType.DMA,
            pltpu.SemaphoreType.DMA,
            pltpu.SemaphoreType.DMA,
            pltpu.SemaphoreType.DMA,
            pltpu.SemaphoreType.DMA,
        ],
    )
    def k(x_hbm, pe_hbm, o_hbm, xbuf, pebuf, obuf,
          ldx0, ldx1, ldp0, ldp1, st0, st1):
        ldx = (ldx0, ldx1)
        ldp = (ldp0, ldp1)
        st = (st0, st1)
        wid = lax.axis_index("s") * NC + lax.axis_index("c")
        base = wid * PER_W
        pe_base = lax.rem(base, S_ELEMS)

        def issue_loads(g, b):
            off = base + g * CH
            poff = pe_base + g * CH
            pltpu.make_async_copy(
                x_hbm.at[pl.ds(off, CH)], xbuf.at[b], ldx[b]).start()
            pltpu.make_async_copy(
                pe_hbm.at[pl.ds(poff, CH)], pebuf.at[b], ldp[b]).start()

        issue_loads(0, 0)
        issue_loads(1, 1)

        def pair_body(p, _):
            for b in range(2):
                g = p * 2 + b
                xb, pb, ob = xbuf.at[b], pebuf.at[b], obuf.at[b]
                pltpu.make_async_copy(
                    x_hbm.at[pl.ds(base, CH)], xb, ldx[b]).wait()
                pltpu.make_async_copy(
                    pe_hbm.at[pl.ds(pe_base, CH)], pb, ldp[b]).wait()

                @pl.when(p >= 1)
                def _():
                    pltpu.make_async_copy(
                        ob, o_hbm.at[pl.ds(base, CH)], st[b]).wait()

                @plsc.parallel_loop(0, CH // 16, unroll=8)
                def vec_body(i):
                    sl = pl.ds(i * 16, 16)
                    ob[sl] = xb[sl] + pb[sl]

                pltpu.make_async_copy(
                    ob, o_hbm.at[pl.ds(base + g * CH, CH)], st[b]).start()

                @pl.when(g + 2 < N_CHUNKS)
                def _():
                    issue_loads(g + 2, b)

            return 0

        lax.fori_loop(0, N_CHUNKS // 2, pair_body, 0)

        for b in range(2):
            pltpu.make_async_copy(
                obuf.at[b], o_hbm.at[pl.ds(base, CH)], st[b]).wait()

    out = k(x_flat, pe_flat)
    return out.reshape(B, S, D)


# speedup vs baseline: 5.1654x; 5.1654x over previous
import jax
import jax.numpy as jnp
from jax.experimental import pallas as pl
from jax.experimental.pallas import tpu as pltpu


def _add_kernel(x_ref, pe_ref, o_ref):
    o_ref[...] = x_ref[...] + pe_ref[...]


def kernel(x, pe):
    B, S, D = x.shape
    BS = 2048
    grid = (S // BS, B)
    return pl.pallas_call(
        _add_kernel,
        grid=grid,
        in_specs=[
            pl.BlockSpec((1, BS, D), lambda i, j: (j, i, 0)),
            pl.BlockSpec((BS, D), lambda i, j: (i, 0)),
        ],
        out_specs=pl.BlockSpec((1, BS, D), lambda i, j: (j, i, 0)),
        out_shape=jax.ShapeDtypeStruct((B, S, D), x.dtype),
        compiler_params=pltpu.CompilerParams(vmem_limit_bytes=128 * 1024 * 1024),
    )(x, pe[:S])
